# Initial kernel scaffold; baseline (speedup 1.0000x reference)
import jax
import jax.numpy as jnp
from jax.experimental import pallas as pl


def kernel(logits, token_ids):
    raise NotImplementedError("write your pallas kernel here")


# speedup vs baseline: 9.3402x; 9.3402x over previous
import jax
import jax.numpy as jnp
import numpy as np
from jax.experimental import pallas as pl
from jax.experimental.pallas import tpu as pltpu

NUM_BEAMS = 4
TOPK = NUM_BEAMS + 1
PEN_UP = np.float32(1.2)
PEN_DOWN = np.float32(1.0) / np.float32(1.2)
VOCAB = 1000000
BATCH = 32
HIST = 200
W = 8192
NB = (VOCAB + W - 1) // W
NEG = np.float32(-np.inf)
IMAX = np.int32(2**31 - 1)


def _body(ids_ref, logits_ref, sc_out, ti_out,
          xs, m_scr, s_scr, tv_scr, ti_scr, ptr_scr):
    b = pl.program_id(0)

    @pl.when(b == 0)
    def _init():
        m_scr[...] = jnp.full((BATCH, 128), NEG, jnp.float32)
        s_scr[...] = jnp.zeros((BATCH, 128), jnp.float32)
        tv_scr[...] = jnp.full((BATCH, 128), NEG, jnp.float32)
        ti_scr[...] = jnp.full((BATCH, 128), IMAX, jnp.int32)

        def _z(r, c):
            ptr_scr[r] = 0
            return c
        jax.lax.fori_loop(0, BATCH, _z, 0)

    xs[...] = logits_ref[...]

    v0 = b * W
    v1 = v0 + W

    def _row(r, c):
        def cond(p):
            pid = ids_ref[r, jnp.minimum(p, HIST - 1)]
            return jnp.logical_and(p < HIST, pid < v1)

        def step(p):
            tok = ids_ref[r, p]
            prev = ids_ref[r, jnp.maximum(p, 1) - 1]
            is_dup = jnp.logical_and(p > 0, tok == prev)

            @pl.when(jnp.logical_not(is_dup))
            def _():
                col = tok - v0
                chi = pl.multiple_of((col // 128) * 128, 128)
                clo = col % 128
                rhi = pl.multiple_of((r // 8) * 8, 8)
                rlo = r % 8
                grp = xs[pl.ds(rhi, 8), pl.ds(chi, 128)]
                lane = jax.lax.broadcasted_iota(jnp.int32, (8, 128), 1)
                sub = jax.lax.broadcasted_iota(jnp.int32, (8, 128), 0)
                pen = jnp.where(grp < 0, grp * PEN_UP, grp * PEN_DOWN)
                sel = jnp.logical_and(lane == clo, sub == rlo)
                xs[pl.ds(rhi, 8), pl.ds(chi, 128)] = jnp.where(sel, pen, grp)
            return p + 1

        ptr_scr[r] = jax.lax.while_loop(cond, step, ptr_scr[r])
        return c
    jax.lax.fori_loop(0, BATCH, _row, 0)

    gcol = v0 + jax.lax.broadcasted_iota(jnp.int32, (BATCH, W), 1)
    x = jnp.where(gcol < VOCAB, xs[...], NEG)

    m_old = m_scr[:, 0:1]
    bm = jnp.max(x, axis=1, keepdims=True)
    m_new = jnp.maximum(m_old, bm)
    s_new = (s_scr[:, 0:1] * jnp.exp(m_old - m_new)
             + jnp.sum(jnp.exp(x - m_new[:, 0:1]), axis=1, keepdims=True))
    m_scr[...] = jnp.broadcast_to(m_new, (BATCH, 128))
    s_scr[...] = jnp.broadcast_to(s_new, (BATCH, 128))

    cur = x
    bvals, bidxs = [], []
    for _ in range(TOPK):
        v = jnp.max(cur, axis=1, keepdims=True)
        hit = cur == v
        idx = jnp.min(jnp.where(hit, gcol, IMAX), axis=1, keepdims=True)
        bvals.append(v)
        bidxs.append(idx)
        cur = jnp.where(gcol == idx, NEG, cur)
    bv = jnp.concatenate(bvals, axis=1)
    bi = jnp.concatenate(bidxs, axis=1)

    cv = jnp.concatenate([tv_scr[:, :TOPK], bv], axis=1)
    ci = jnp.concatenate([ti_scr[:, :TOPK], bi], axis=1)
    nvals, nidxs = [], []
    for _ in range(TOPK):
        v = jnp.max(cv, axis=1, keepdims=True)
        hit = cv == v
        idx = jnp.min(jnp.where(hit, ci, IMAX), axis=1, keepdims=True)
        nvals.append(v)
        nidxs.append(idx)
        cv = jnp.where(ci == idx, NEG, cv)
    tv_new = jnp.concatenate(nvals, axis=1)
    ti_new = jnp.concatenate(nidxs, axis=1)
    pad = ((0, 0), (0, 128 - TOPK))
    tv_scr[...] = jnp.pad(tv_new, pad, constant_values=NEG)
    ti_scr[...] = jnp.pad(ti_new, pad, constant_values=IMAX)

    @pl.when(b == NB - 1)
    def _fin():
        lse = jnp.log(s_new)
        opad = ((0, 0), (0, 8 - TOPK))
        sc_out[...] = jnp.pad((tv_new - m_new) - lse, opad, constant_values=NEG)
        ti_out[...] = jnp.pad(ti_new, opad, constant_values=IMAX)


_call = pl.pallas_call(
    _body,
    grid=(NB,),
    in_specs=[
        pl.BlockSpec(memory_space=pltpu.SMEM),
        pl.BlockSpec((BATCH, W), lambda b: (0, b)),
    ],
    out_specs=[
        pl.BlockSpec((BATCH, 8), lambda b: (0, 0)),
        pl.BlockSpec((BATCH, 8), lambda b: (0, 0)),
    ],
    out_shape=[
        jax.ShapeDtypeStruct((BATCH, 8), jnp.float32),
        jax.ShapeDtypeStruct((BATCH, 8), jnp.int32),
    ],
    scratch_shapes=[
        pltpu.VMEM((BATCH, W), jnp.float32),
        pltpu.VMEM((BATCH, 128), jnp.float32),
        pltpu.VMEM((BATCH, 128), jnp.float32),
        pltpu.VMEM((BATCH, 128), jnp.float32),
        pltpu.VMEM((BATCH, 128), jnp.int32),
        pltpu.SMEM((BATCH,), jnp.int32),
    ],
    compiler_params=pltpu.CompilerParams(
        dimension_semantics=("arbitrary",),
    ),
)


def kernel(logits, token_ids):
    ids_sorted = jnp.sort(token_ids, axis=1)
    sc, ti = _call(ids_sorted, logits)
    return sc[:, :TOPK], ti[:, :TOPK]
